# baseline (speedup 1.0000x reference)
import functools

import jax
import jax.numpy as jnp
from jax import lax
from jax.experimental import pallas as pl
from jax.experimental.pallas import tpu as pltpu
from jax.experimental.pallas import tpu_sc as plsc

BATCH = 4096
SAMPLE_NUM = 20
WALK_LENGTH = 20
DIM = 32
NUM_USERS = 1000000
PER_B = SAMPLE_NUM * WALK_LENGTH
SCALE = 1.0 / WALK_LENGTH

NUM_CORES = 2
NUM_SUBCORES = 16
NUM_WORKERS = NUM_CORES * NUM_SUBCORES
B_PER_W = BATCH // NUM_WORKERS

ROWS = 200
NUM_CHUNKS = NUM_USERS // ROWS
FULL_ROUNDS = NUM_CHUNKS // NUM_WORKERS
UNIFORM_END = FULL_ROUNDS * NUM_WORKERS
LEFTOVER = NUM_CHUNKS - UNIFORM_END
DEPAD_UNROLL = 8

GATHER_CHUNKS = ((0, 128), (128, 128), (256, 128), (384, 16))
UNROLL = 8


def _depad_body(tab_hbm, out_hbm, buf_v, flat_v, isem0, isem1, osem0, osem1):
    cid = lax.axis_index("c")
    sid = lax.axis_index("s")
    wid = sid * NUM_CORES + cid

    isems = (isem0, isem1)
    osems = (osem0, osem1)

    def in_desc(slot, k):
        return pltpu.make_async_copy(
            tab_hbm.at[pl.ds(k * ROWS, ROWS), :], buf_v.at[slot],
            isems[slot])

    def out_desc(slot, k):
        return pltpu.make_async_copy(
            flat_v.at[slot],
            out_hbm.at[pl.ds(k * (ROWS * DIM), ROWS * DIM)], osems[slot])

    def compact(slot):
        def body(r8, carry):
            for u in range(DEPAD_UNROLL):
                r = r8 * DEPAD_UNROLL + u
                for h in range(2):
                    flat_v[slot, pl.ds(r * DIM + 16 * h, 16)] = (
                        buf_v[slot, r, pl.ds(16 * h, 16)])
            return carry

        lax.fori_loop(0, ROWS // DEPAD_UNROLL, body, 0)

    in_desc(0, wid).start()
    in_desc(1, NUM_WORKERS + wid).start()

    def rounds(t2, carry):
        for s in range(2):
            t = 2 * t2 + s
            k = t * NUM_WORKERS + wid
            in_desc(s, k).wait()

            @pl.when(t >= 2)
            def _():
                out_desc(s, k).wait()
            compact(s)
            out_desc(s, k).start()
            nk = k + 2 * NUM_WORKERS
            @pl.when(nk < UNIFORM_END)
            def _():
                in_desc(s, nk).start()
        return carry

    lax.fori_loop(0, FULL_ROUNDS // 2, rounds, 0)

    out_desc(0, 0).wait()
    out_desc(1, 0).wait()

    @pl.when(wid < LEFTOVER)
    def _():
        kl = UNIFORM_END + wid
        in_desc(0, kl).start()
        in_desc(0, kl).wait()
        compact(0)
        out_desc(0, kl).start()
        out_desc(0, kl).wait()


def _walk_body(walk_hbm, table_hbm, out_hbm, idx_v, rows_v, out_v, sem0, sem1):
    cid = lax.axis_index("c")
    sid = lax.axis_index("s")
    wid = sid * NUM_CORES + cid
    base_b = wid * B_PER_W

    pltpu.sync_copy(walk_hbm.at[pl.ds(base_b, B_PER_W)], idx_v)

    sems = (sem0, sem1)

    def gather_descs(slot, b):
        descs = []
        for off, n in GATHER_CHUNKS:
            descs.append(pltpu.make_async_copy(
                table_hbm.at[idx_v.at[b, pl.ds(off, n)]],
                rows_v.at[slot, pl.ds(off, n)],
                sems[slot]))
        return descs

    def start_gathers(slot, b):
        for d in gather_descs(slot, b):
            d.start()

    def wait_gathers(slot, b):
        for d in gather_descs(slot, b):
            d.wait()

    def accumulate(slot, b):
        zero = jnp.zeros((16,), jnp.float32)

        def body(r, carry):
            l0, l1, h0, h1 = carry
            base = r * UNROLL
            for j in range(UNROLL):
                lo = rows_v[slot, base + j, pl.ds(0, 16)]
                hi = rows_v[slot, base + j, pl.ds(16, 16)]
                if j % 2 == 0:
                    l0 = l0 + lo
                    h0 = h0 + hi
                else:
                    l1 = l1 + lo
                    h1 = h1 + hi
            return l0, l1, h0, h1

        l0, l1, h0, h1 = lax.fori_loop(
            0, PER_B // UNROLL, body, (zero, zero, zero, zero))
        out_v[b, pl.ds(0, 16)] = (l0 + l1) * SCALE
        out_v[b, pl.ds(16, 16)] = (h0 + h1) * SCALE

    start_gathers(0, 0)

    def outer(g, carry):
        for slot in range(2):
            b = 2 * g + slot
            nb = jnp.minimum(b + 1, B_PER_W - 1)
            wait_gathers(slot, b)
            start_gathers(1 - slot, nb)
            accumulate(slot, b)
        return carry

    lax.fori_loop(0, B_PER_W // 2, outer, 0)

    wait_gathers(0, B_PER_W - 1)

    pltpu.sync_copy(out_v, out_hbm.at[pl.ds(base_b, B_PER_W)])


def _sc_mesh():
    return plsc.VectorSubcoreMesh(core_axis_name="c", subcore_axis_name="s")


@jax.jit
def _walk_aggregate(walk2d, user_table):
    depad_fn = functools.partial(
        pl.kernel,
        out_type=jax.ShapeDtypeStruct((NUM_USERS * DIM,), jnp.float32),
        mesh=_sc_mesh(),
        scratch_types=[
            pltpu.VMEM((2, ROWS, DIM), jnp.float32),
            pltpu.VMEM((2, ROWS * DIM), jnp.float32),
            pltpu.SemaphoreType.DMA,
            pltpu.SemaphoreType.DMA,
            pltpu.SemaphoreType.DMA,
            pltpu.SemaphoreType.DMA,
        ],
        compiler_params=pltpu.CompilerParams(use_tc_tiling_on_sc=True),
    )(_depad_body)
    table_rm = depad_fn(user_table).reshape(NUM_USERS, DIM)

    gather_fn = functools.partial(
        pl.kernel,
        out_type=jax.ShapeDtypeStruct((BATCH, DIM), jnp.float32),
        mesh=_sc_mesh(),
        scratch_types=[
            pltpu.VMEM((B_PER_W, PER_B), jnp.int32),
            pltpu.VMEM((2, PER_B, DIM), jnp.float32),
            pltpu.VMEM((B_PER_W, DIM), jnp.float32),
            pltpu.SemaphoreType.DMA,
            pltpu.SemaphoreType.DMA,
        ],
        compiler_params=pltpu.CompilerParams(use_tc_tiling_on_sc=False),
    )(_walk_body)
    return gather_fn(walk2d, table_rm)


def kernel(walk_nodes, predict_times, user_table):
    del predict_times
    walk2d = walk_nodes.reshape(BATCH, PER_B)
    return _walk_aggregate(walk2d, user_table)


# speedup vs baseline: 1.0108x; 1.0108x over previous
import functools

import jax
import jax.numpy as jnp
from jax import lax
from jax.experimental import pallas as pl
from jax.experimental.pallas import tpu as pltpu
from jax.experimental.pallas import tpu_sc as plsc

BATCH = 4096
SAMPLE_NUM = 20
WALK_LENGTH = 20
DIM = 32
NUM_USERS = 1000000
PER_B = SAMPLE_NUM * WALK_LENGTH
SCALE = 1.0 / WALK_LENGTH

NUM_CORES = 2
NUM_SUBCORES = 16
NUM_WORKERS = NUM_CORES * NUM_SUBCORES
B_PER_W = BATCH // NUM_WORKERS

ROWS = 400
NUM_CHUNKS = NUM_USERS // ROWS
FULL_ROUNDS = NUM_CHUNKS // NUM_WORKERS
UNIFORM_END = FULL_ROUNDS * NUM_WORKERS
LEFTOVER = NUM_CHUNKS - UNIFORM_END
DEPAD_UNROLL = 8

GATHER_CHUNKS = ((0, 128), (128, 128), (256, 128), (384, 16))
UNROLL = 8


def _depad_body(tab_hbm, out_hbm, buf_v, flat_v, isem0, isem1, osem0, osem1):
    cid = lax.axis_index("c")
    sid = lax.axis_index("s")
    wid = sid * NUM_CORES + cid

    isems = (isem0, isem1)
    osems = (osem0, osem1)

    def in_desc(slot, k):
        return pltpu.make_async_copy(
            tab_hbm.at[pl.ds(k * ROWS, ROWS), :], buf_v.at[slot],
            isems[slot])

    def out_desc(slot, k):
        return pltpu.make_async_copy(
            flat_v.at[0],
            out_hbm.at[pl.ds(k * (ROWS * DIM), ROWS * DIM)], osems[slot])

    def compact(slot):
        def body(r8, carry):
            for u in range(DEPAD_UNROLL):
                r = r8 * DEPAD_UNROLL + u
                for h in range(2):
                    flat_v[0, pl.ds(r * DIM + 16 * h, 16)] = (
                        buf_v[slot, r, pl.ds(16 * h, 16)])
            return carry

        lax.fori_loop(0, ROWS // DEPAD_UNROLL, body, 0)

    in_desc(0, wid).start()
    in_desc(1, NUM_WORKERS + wid).start()

    def rounds(t2, carry):
        for s in range(2):
            t = 2 * t2 + s
            k = t * NUM_WORKERS + wid
            in_desc(s, k).wait()

            @pl.when(t >= 1)
            def _():
                out_desc(0, k).wait()
            compact(s)
            out_desc(0, k).start()
            nk = k + 2 * NUM_WORKERS
            @pl.when(nk < UNIFORM_END)
            def _():
                in_desc(s, nk).start()
        return carry

    lax.fori_loop(0, FULL_ROUNDS // 2, rounds, 0)

    out_desc(0, 0).wait()

    @pl.when(wid < LEFTOVER)
    def _():
        kl = UNIFORM_END + wid
        in_desc(0, kl).start()
        in_desc(0, kl).wait()
        compact(0)
        out_desc(0, kl).start()
        out_desc(0, kl).wait()


def _walk_body(walk_hbm, table_hbm, out_hbm, idx_v, rows_v, out_v, sem0, sem1):
    cid = lax.axis_index("c")
    sid = lax.axis_index("s")
    wid = sid * NUM_CORES + cid
    base_b = wid * B_PER_W

    pltpu.sync_copy(walk_hbm.at[pl.ds(base_b, B_PER_W)], idx_v)

    sems = (sem0, sem1)

    def gather_descs(slot, b):
        descs = []
        for off, n in GATHER_CHUNKS:
            descs.append(pltpu.make_async_copy(
                table_hbm.at[idx_v.at[b, pl.ds(off, n)]],
                rows_v.at[slot, pl.ds(off, n)],
                sems[slot]))
        return descs

    def start_gathers(slot, b):
        for d in gather_descs(slot, b):
            d.start()

    def wait_gathers(slot, b):
        for d in gather_descs(slot, b):
            d.wait()

    def accumulate(slot, b):
        zero = jnp.zeros((16,), jnp.float32)

        def body(r, carry):
            l0, l1, h0, h1 = carry
            base = r * UNROLL
            for j in range(UNROLL):
                lo = rows_v[slot, base + j, pl.ds(0, 16)]
                hi = rows_v[slot, base + j, pl.ds(16, 16)]
                if j % 2 == 0:
                    l0 = l0 + lo
                    h0 = h0 + hi
                else:
                    l1 = l1 + lo
                    h1 = h1 + hi
            return l0, l1, h0, h1

        l0, l1, h0, h1 = lax.fori_loop(
            0, PER_B // UNROLL, body, (zero, zero, zero, zero))
        out_v[b, pl.ds(0, 16)] = (l0 + l1) * SCALE
        out_v[b, pl.ds(16, 16)] = (h0 + h1) * SCALE

    start_gathers(0, 0)

    def outer(g, carry):
        for slot in range(2):
            b = 2 * g + slot
            nb = jnp.minimum(b + 1, B_PER_W - 1)
            wait_gathers(slot, b)
            start_gathers(1 - slot, nb)
            accumulate(slot, b)
        return carry

    lax.fori_loop(0, B_PER_W // 2, outer, 0)

    wait_gathers(0, B_PER_W - 1)

    pltpu.sync_copy(out_v, out_hbm.at[pl.ds(base_b, B_PER_W)])


def _sc_mesh():
    return plsc.VectorSubcoreMesh(core_axis_name="c", subcore_axis_name="s")


@jax.jit
def _walk_aggregate(walk2d, user_table):
    depad_fn = functools.partial(
        pl.kernel,
        out_type=jax.ShapeDtypeStruct((NUM_USERS * DIM,), jnp.float32),
        mesh=_sc_mesh(),
        scratch_types=[
            pltpu.VMEM((2, ROWS, DIM), jnp.float32),
            pltpu.VMEM((1, ROWS * DIM), jnp.float32),
            pltpu.SemaphoreType.DMA,
            pltpu.SemaphoreType.DMA,
            pltpu.SemaphoreType.DMA,
            pltpu.SemaphoreType.DMA,
        ],
        compiler_params=pltpu.CompilerParams(use_tc_tiling_on_sc=True),
    )(_depad_body)
    table_rm = depad_fn(user_table).reshape(NUM_USERS, DIM)

    gather_fn = functools.partial(
        pl.kernel,
        out_type=jax.ShapeDtypeStruct((BATCH, DIM), jnp.float32),
        mesh=_sc_mesh(),
        scratch_types=[
            pltpu.VMEM((B_PER_W, PER_B), jnp.int32),
            pltpu.VMEM((2, PER_B, DIM), jnp.float32),
            pltpu.VMEM((B_PER_W, DIM), jnp.float32),
            pltpu.SemaphoreType.DMA,
            pltpu.SemaphoreType.DMA,
        ],
        compiler_params=pltpu.CompilerParams(use_tc_tiling_on_sc=False),
    )(_walk_body)
    return gather_fn(walk2d, table_rm)


def kernel(walk_nodes, predict_times, user_table):
    del predict_times
    walk2d = walk_nodes.reshape(BATCH, PER_B)
    return _walk_aggregate(walk2d, user_table)


# speedup vs baseline: 1.0872x; 1.0756x over previous
import functools

import jax
import jax.numpy as jnp
from jax import lax
from jax.experimental import pallas as pl
from jax.experimental.pallas import tpu as pltpu
from jax.experimental.pallas import tpu_sc as plsc

BATCH = 4096
SAMPLE_NUM = 20
WALK_LENGTH = 20
DIM = 32
PER_B = SAMPLE_NUM * WALK_LENGTH
SCALE = 1.0 / WALK_LENGTH

NUM_CORES = 2
NUM_SUBCORES = 16
NUM_WORKERS = NUM_CORES * NUM_SUBCORES
B_PER_W = BATCH // NUM_WORKERS

GATHER_CHUNKS = ((0, 128), (128, 128), (256, 128), (384, 16))
UNROLL = 8


def _walk_body(walk_hbm, table_hbm, out_hbm, idx_v, rows_v, out_v, sem0, sem1):
    cid = lax.axis_index("c")
    sid = lax.axis_index("s")
    wid = sid * NUM_CORES + cid
    base_b = wid * B_PER_W

    pltpu.sync_copy(walk_hbm.at[pl.ds(base_b, B_PER_W)], idx_v)

    sems = (sem0, sem1)

    def gather_descs(slot, b):
        descs = []
        for off, n in GATHER_CHUNKS:
            descs.append(pltpu.make_async_copy(
                table_hbm.at[idx_v.at[b, pl.ds(off, n)]],
                rows_v.at[slot, pl.ds(off, n)],
                sems[slot]))
        return descs

    def start_gathers(slot, b):
        for d in gather_descs(slot, b):
            d.start()

    def wait_gathers(slot, b):
        for d in gather_descs(slot, b):
            d.wait()

    def accumulate(slot, b):
        zero = jnp.zeros((16,), jnp.float32)

        def body(r, carry):
            l0, l1, h0, h1 = carry
            base = r * UNROLL
            for j in range(UNROLL):
                lo = rows_v[slot, base + j, pl.ds(0, 16)]
                hi = rows_v[slot, base + j, pl.ds(16, 16)]
                if j % 2 == 0:
                    l0 = l0 + lo
                    h0 = h0 + hi
                else:
                    l1 = l1 + lo
                    h1 = h1 + hi
            return l0, l1, h0, h1

        l0, l1, h0, h1 = lax.fori_loop(
            0, PER_B // UNROLL, body, (zero, zero, zero, zero))
        out_v[b, pl.ds(0, 16)] = (l0 + l1) * SCALE
        out_v[b, pl.ds(16, 16)] = (h0 + h1) * SCALE

    start_gathers(0, 0)

    def outer(g, carry):
        for slot in range(2):
            b = 2 * g + slot
            nb = jnp.minimum(b + 1, B_PER_W - 1)
            wait_gathers(slot, b)
            start_gathers(1 - slot, nb)
            accumulate(slot, b)
        return carry

    lax.fori_loop(0, B_PER_W // 2, outer, 0)

    wait_gathers(0, B_PER_W - 1)

    pltpu.sync_copy(out_v, out_hbm.at[pl.ds(base_b, B_PER_W)])


@functools.partial(jax.jit, static_argnames=())
def _walk_aggregate(walk2d, user_table):
    mesh = plsc.VectorSubcoreMesh(core_axis_name="c", subcore_axis_name="s")
    f = functools.partial(
        pl.kernel,
        out_type=jax.ShapeDtypeStruct((BATCH, DIM), jnp.float32),
        mesh=mesh,
        scratch_types=[
            pltpu.VMEM((B_PER_W, PER_B), jnp.int32),
            pltpu.VMEM((2, PER_B, DIM), jnp.float32),
            pltpu.VMEM((B_PER_W, DIM), jnp.float32),
            pltpu.SemaphoreType.DMA,
            pltpu.SemaphoreType.DMA,
        ],
        compiler_params=pltpu.CompilerParams(use_tc_tiling_on_sc=False),
    )(_walk_body)
    return f(walk2d, user_table)


def kernel(walk_nodes, predict_times, user_table):
    del predict_times
    walk2d = walk_nodes.reshape(BATCH, PER_B)
    return _walk_aggregate(walk2d, user_table)
